# Initial kernel scaffold; baseline (speedup 1.0000x reference)
import jax
import jax.numpy as jnp
from jax.experimental import pallas as pl


def kernel(x, edge_index, W0, W1, W2, bias, prelu_weight, bn_weight, bn_bias):
    raise NotImplementedError("write your pallas kernel here")


# speedup vs baseline: 12.1509x; 12.1509x over previous
import functools

import jax
import jax.numpy as jnp
from jax import lax
from jax.experimental import pallas as pl
from jax.experimental.pallas import tpu as pltpu
from jax.experimental.pallas import tpu_sc as plsc

N = 10000
D = 128
E = 320000

NC = 2
NS = 16
NW = NC * NS
CHUNK = 128
CH = 80
EPW = CH * CHUNK
E_PAD = NW * EPW
N_PAD = 10240
ROWS_PER_TILE = N_PAD // NS
BLK = 2000

_mesh = plsc.VectorSubcoreMesh(core_axis_name="c", subcore_axis_name="s")


@functools.partial(
    pl.kernel,
    out_type=(
        jax.ShapeDtypeStruct((NW, CH, CHUNK), jnp.int32),
        jax.ShapeDtypeStruct((NW, CH, CHUNK), jnp.int32),
    ),
    mesh=_mesh,
    scratch_types=[
        pltpu.VMEM((CH, CHUNK), jnp.int32),
        pltpu.VMEM((CH, CHUNK), jnp.int32),
        pltpu.VMEM((CH, CHUNK), jnp.int32),
        pltpu.VMEM((CH, CHUNK), jnp.int32),
    ],
)
def _sc_prep(src_hbm, dst_hbm, src2_hbm, dst2_hbm, sv, dv, s2, d2):
    cid = lax.axis_index("c")
    tid = lax.axis_index("s")
    wid = cid * NS + tid

    pltpu.sync_copy(src_hbm.at[wid], sv)
    pltpu.sync_copy(dst_hbm.at[wid], dv)
    iota = lax.iota(jnp.int32, 16)

    @pl.loop(0, CH)
    def _(ci):
        @pl.loop(0, CHUNK // 16)
        def _(j):
            s16 = sv[ci, pl.ds(j * 16, 16)]
            d16 = dv[ci, pl.ds(j * 16, 16)]
            tv = iota + (N + 16 * ((ci * 8 + j) % 15))
            eq = s16 == d16
            s2[ci, pl.ds(j * 16, 16)] = jnp.where(eq, tv, s16)
            d2[ci, pl.ds(j * 16, 16)] = jnp.where(eq, tv, d16)

    pltpu.sync_copy(s2, src2_hbm.at[wid])
    pltpu.sync_copy(d2, dst2_hbm.at[wid])


@functools.partial(
    pl.kernel,
    out_type=jax.ShapeDtypeStruct((NC, N_PAD, D), jnp.float32),
    mesh=_mesh,
    scratch_types=[
        pltpu.VMEM((CH, CHUNK), jnp.int32),
        pltpu.VMEM((CH, CHUNK), jnp.int32),
        pltpu.VMEM((CHUNK, D), jnp.float32),
        pltpu.VMEM_SHARED((N_PAD, D), jnp.float32),
    ],
)
def _sc_prop(src_hbm, dst2_hbm, g_hbm, out_hbm, sv, d2, rows, acc):
    cid = lax.axis_index("c")
    tid = lax.axis_index("s")
    wid = cid * NS + tid

    @pl.loop(0, CHUNK)
    def _(i):
        @pl.loop(0, D // 16)
        def _(j):
            rows[i, pl.ds(j * 16, 16)] = jnp.zeros((16,), jnp.float32)

    @pl.loop(0, ROWS_PER_TILE // CHUNK)
    def _(k):
        pltpu.sync_copy(rows, acc.at[pl.ds(tid * ROWS_PER_TILE + k * CHUNK, CHUNK)])

    pltpu.sync_copy(src_hbm.at[wid], sv)
    pltpu.sync_copy(dst2_hbm.at[wid], d2)
    plsc.subcore_barrier()

    @pl.loop(0, CH)
    def _(ci):
        pltpu.sync_copy(g_hbm.at[sv.at[ci]], rows)
        pltpu.sync_copy(rows, acc.at[d2.at[ci]], add=True)

    plsc.subcore_barrier()
    pltpu.sync_copy(
        acc.at[pl.ds(tid * ROWS_PER_TILE, ROWS_PER_TILE)],
        out_hbm.at[cid, pl.ds(tid * ROWS_PER_TILE, ROWS_PER_TILE)],
    )


def _dis_of(deg_ref, nrows):
    deg = deg_ref[0, :nrows, 0:1] + deg_ref[1, :nrows, 0:1]
    return jnp.where(deg > 0, lax.rsqrt(jnp.where(deg > 0, deg, 1.0)), 0.0)


def _tc_scale1_body(deg_ref, x_ref, g0_ref):
    g0_ref[...] = _dis_of(deg_ref, BLK) * x_ref[...]


_tc_scale1 = pl.pallas_call(
    _tc_scale1_body,
    grid=(N // BLK,),
    in_specs=[
        pl.BlockSpec((NC, BLK, D), lambda i: (0, i, 0)),
        pl.BlockSpec((BLK, D), lambda i: (i, 0)),
    ],
    out_specs=pl.BlockSpec((BLK, D), lambda i: (i, 0)),
    out_shape=jax.ShapeDtypeStruct((N, D), jnp.float32),
)


def _tc_scale2_body(deg_ref, s1_ref, tx1_ref, g1_ref):
    dis = _dis_of(deg_ref, BLK)
    s1 = s1_ref[0] + s1_ref[1]
    tx1 = -dis * s1
    tx1_ref[...] = tx1
    g1_ref[...] = dis * tx1


_tc_scale2 = pl.pallas_call(
    _tc_scale2_body,
    grid=(N // BLK,),
    in_specs=[
        pl.BlockSpec((NC, BLK, D), lambda i: (0, i, 0)),
        pl.BlockSpec((NC, BLK, D), lambda i: (0, i, 0)),
    ],
    out_specs=[
        pl.BlockSpec((BLK, D), lambda i: (i, 0)),
        pl.BlockSpec((BLK, D), lambda i: (i, 0)),
    ],
    out_shape=[
        jax.ShapeDtypeStruct((N, D), jnp.float32),
        jax.ShapeDtypeStruct((N, D), jnp.float32),
    ],
)


def _tc_pre_body(x_ref, tx1_ref, s2_ref, deg_ref, w0_ref, w1_ref, w2_ref,
                 b_ref, pw_ref, h_ref, st_ref):
    dis = _dis_of(deg_ref, BLK)
    x = x_ref[...]
    s2 = s2_ref[0] + s2_ref[1]
    tx2 = (-2.0) * dis * s2 - x
    dot = functools.partial(
        jnp.dot, precision=lax.Precision.HIGHEST,
        preferred_element_type=jnp.float32,
    )
    h = (dot(x, w0_ref[...]) + dot(tx1_ref[...], w1_ref[...])
         + dot(tx2, w2_ref[...]) + b_ref[...])
    h = jnp.where(h >= 0.0, h, pw_ref[0, 0] * h)
    h_ref[...] = h
    st_ref[0, 0, :] = jnp.sum(h, axis=0)
    st_ref[0, 1, :] = jnp.sum(h * h, axis=0)


_tc_pre = pl.pallas_call(
    _tc_pre_body,
    grid=(N // BLK,),
    in_specs=[
        pl.BlockSpec((BLK, D), lambda i: (i, 0)),
        pl.BlockSpec((BLK, D), lambda i: (i, 0)),
        pl.BlockSpec((NC, BLK, D), lambda i: (0, i, 0)),
        pl.BlockSpec((NC, BLK, D), lambda i: (0, i, 0)),
        pl.BlockSpec((D, D), lambda i: (0, 0)),
        pl.BlockSpec((D, D), lambda i: (0, 0)),
        pl.BlockSpec((D, D), lambda i: (0, 0)),
        pl.BlockSpec((1, D), lambda i: (0, 0)),
        pl.BlockSpec((1, 1), lambda i: (0, 0)),
    ],
    out_specs=[
        pl.BlockSpec((BLK, D), lambda i: (i, 0)),
        pl.BlockSpec((1, 2, D), lambda i: (i, 0, 0)),
    ],
    out_shape=[
        jax.ShapeDtypeStruct((N, D), jnp.float32),
        jax.ShapeDtypeStruct((N // BLK, 2, D), jnp.float32),
    ],
)


def _tc_bn_body(h_ref, st_ref, bnw_ref, bnb_ref, o_ref):
    s = jnp.sum(st_ref[:, 0, :], axis=0, keepdims=True)
    s2 = jnp.sum(st_ref[:, 1, :], axis=0, keepdims=True)
    mean = s * (1.0 / N)
    var = s2 * (1.0 / N) - mean * mean
    scale = lax.rsqrt(var + 1e-5) * bnw_ref[...]
    o_ref[...] = (h_ref[...] - mean) * scale + bnb_ref[...]


_tc_bn = pl.pallas_call(
    _tc_bn_body,
    grid=(N // BLK,),
    in_specs=[
        pl.BlockSpec((BLK, D), lambda i: (i, 0)),
        pl.BlockSpec((N // BLK, 2, D), lambda i: (0, 0, 0)),
        pl.BlockSpec((1, D), lambda i: (0, 0)),
        pl.BlockSpec((1, D), lambda i: (0, 0)),
    ],
    out_specs=pl.BlockSpec((BLK, D), lambda i: (i, 0)),
    out_shape=jax.ShapeDtypeStruct((N, D), jnp.float32),
)


def kernel(x, edge_index, W0, W1, W2, bias, prelu_weight, bn_weight, bn_bias):
    src = edge_index[0]
    dst = edge_index[1]
    pad = jnp.arange(E_PAD - E, dtype=jnp.int32)
    src_p = jnp.concatenate([src, pad]).reshape(NW, CH, CHUNK)
    dst_p = jnp.concatenate([dst, pad]).reshape(NW, CH, CHUNK)

    src2, dst2 = _sc_prep(src_p, dst_p)
    ones_n = jnp.ones((N, D), jnp.float32)
    deg_parts = _sc_prop(src_p, src2, ones_n)
    g0 = _tc_scale1(deg_parts, x)
    s1_parts = _sc_prop(src_p, dst2, g0)
    tx1, g1 = _tc_scale2(deg_parts, s1_parts)
    s2_parts = _sc_prop(src_p, dst2, g1)
    h, stats = _tc_pre(
        x, tx1, s2_parts, deg_parts, W0, W1, W2,
        bias.reshape(1, D), prelu_weight.reshape(1, 1),
    )
    return _tc_bn(h, stats, bn_weight.reshape(1, D), bn_bias.reshape(1, D))
